# Initial kernel scaffold; baseline (speedup 1.0000x reference)
import jax
import jax.numpy as jnp
from jax.experimental import pallas as pl


def kernel(focal_input, context_input, cooc_count, focal_table, context_table, focal_biases, context_biases):
    raise NotImplementedError("write your pallas kernel here")


# speedup vs baseline: 1.5643x; 1.5643x over previous
import functools

import jax
import jax.numpy as jnp
from jax import lax
from jax.experimental import pallas as pl
from jax.experimental.pallas import tpu as pltpu
from jax.experimental.pallas import tpu_sc as plsc

NC = 2
NS = 16
NW = NC * NS
L = 16

X_MAX = 100.0
ALPHA = 0.75
LN2 = 0.6931471805599453
W_BIAS = ALPHA * 4.605170185988091


def _ln(x):
    bits = plsc.bitcast(x, jnp.int32)
    e = lax.shift_right_arithmetic(bits, 23) - 127
    mbits = (bits & 0x007FFFFF) | 0x3F800000
    m = plsc.bitcast(mbits, jnp.float32)
    big = m > 1.41421356
    m = jnp.where(big, m * 0.5, m)
    ef = e.astype(jnp.float32) + jnp.where(big, 1.0, 0.0)
    s = (m - 1.0) / (m + 1.0)
    z = s * s
    p = jnp.float32(1.0 / 9.0)
    p = p * z + jnp.float32(1.0 / 7.0)
    p = p * z + jnp.float32(0.2)
    p = p * z + jnp.float32(1.0 / 3.0)
    p = p * z + jnp.float32(1.0)
    return ef * jnp.float32(LN2) + 2.0 * s * p


def _make_glove(B, E):
    assert B % NW == 0 and E % L == 0
    b_per_w = B // NW
    CHUNK = 128
    n_chunks = b_per_w // CHUNK
    EJ = E // L

    mesh = plsc.VectorSubcoreMesh(
        core_axis_name="c", subcore_axis_name="s", num_cores=NC, num_subcores=NS
    )

    @functools.partial(
        pl.kernel,
        out_type=jax.ShapeDtypeStruct((NW, L), jnp.float32),
        mesh=mesh,
        compiler_params=pltpu.CompilerParams(needs_layout_passes=False),
        scratch_types=[
            pltpu.VMEM((b_per_w,), jnp.int32),
            pltpu.VMEM((b_per_w,), jnp.int32),
            pltpu.VMEM((b_per_w,), jnp.float32),
            pltpu.VMEM((2, CHUNK, E), jnp.float32),
            pltpu.VMEM((2, CHUNK, E), jnp.float32),
            pltpu.VMEM((2, CHUNK), jnp.float32),
            pltpu.VMEM((2, CHUNK), jnp.float32),
            pltpu.VMEM((L * L,), jnp.float32),
            pltpu.VMEM((L,), jnp.float32),
            pltpu.SemaphoreType.DMA,
            pltpu.SemaphoreType.DMA,
        ],
    )
    def glove(fidx_hbm, cidx_hbm, cooc_hbm, ftab_hbm, ctab_hbm, fb_hbm, cb_hbm,
              out_hbm, fidx_v, cidx_v, cooc_v, frows_v, crows_v, fbv, cbv,
              pv, accv, sem0, sem1):
        wid = lax.axis_index("s") * NC + lax.axis_index("c")
        base = wid * b_per_w

        pltpu.sync_copy(fidx_hbm.at[pl.ds(base, b_per_w)], fidx_v)
        pltpu.sync_copy(cidx_hbm.at[pl.ds(base, b_per_w)], cidx_v)
        pltpu.sync_copy(cooc_hbm.at[pl.ds(base, b_per_w)], cooc_v)

        sems = (sem0, sem1)

        def fire(k):
            buf = k % 2
            sem = sems[buf]
            fi = fidx_v.at[pl.ds(k * CHUNK, CHUNK)]
            ci = cidx_v.at[pl.ds(k * CHUNK, CHUNK)]
            return [
                pltpu.async_copy(ftab_hbm.at[fi], frows_v.at[buf], sem),
                pltpu.async_copy(ctab_hbm.at[ci], crows_v.at[buf], sem),
                pltpu.async_copy(fb_hbm.at[fi], fbv.at[buf], sem),
                pltpu.async_copy(cb_hbm.at[ci], cbv.at[buf], sem),
            ]

        rid16 = lax.iota(jnp.int32, 16) * L
        inflight = fire(0)
        acc = jnp.zeros((L,), jnp.float32)

        for k in range(n_chunks):
            buf = k % 2
            for cp in inflight:
                cp.wait()
            if k + 1 < n_chunks:
                inflight = fire(k + 1)

            def group(g, acc):
                pbase = g * L
                for p in range(L):
                    row = pbase + p
                    prod = (frows_v[buf, row, pl.ds(0, L)]
                            * crows_v[buf, row, pl.ds(0, L)])
                    for j in range(1, EJ):
                        prod += (frows_v[buf, row, pl.ds(j * L, L)]
                                 * crows_v[buf, row, pl.ds(j * L, L)])
                    pv[pl.ds(p * L, L)] = prod
                dots = plsc.load_gather(pv, [rid16])
                for j in range(1, L):
                    dots += plsc.load_gather(pv, [rid16 + j])
                f_b = fbv[buf, pl.ds(pbase, L)]
                c_b = cbv[buf, pl.ds(pbase, L)]
                x = cooc_v[pl.ds(k * CHUNK + pbase, L)]
                lnx = _ln(x)
                d = dots + f_b + c_b - lnx
                w = jnp.minimum(jnp.exp(ALPHA * lnx - W_BIAS), 1.0)
                return acc + w * d * d

            acc = lax.fori_loop(0, CHUNK // L, group, acc)

        accv[:] = acc
        pltpu.sync_copy(accv, out_hbm.at[wid])

    return glove


def kernel(focal_input, context_input, cooc_count, focal_table, context_table,
           focal_biases, context_biases):
    B = focal_input.shape[0]
    E = focal_table.shape[1]
    glove = _make_glove(B, E)
    partials = glove(
        focal_input.astype(jnp.int32), context_input.astype(jnp.int32),
        cooc_count, focal_table, context_table, focal_biases, context_biases)
    return jnp.sum(partials) / B


# speedup vs baseline: 1.6522x; 1.0562x over previous
import functools

import jax
import jax.numpy as jnp
from jax import lax
from jax.experimental import pallas as pl
from jax.experimental.pallas import tpu as pltpu
from jax.experimental.pallas import tpu_sc as plsc

NC = 2
NS = 16
NW = NC * NS
L = 16

X_MAX = 100.0
ALPHA = 0.75
LN2 = 0.6931471805599453
W_BIAS = ALPHA * 4.605170185988091


def _ln(x):
    bits = plsc.bitcast(x, jnp.int32)
    e = lax.shift_right_arithmetic(bits, 23) - 127
    mbits = (bits & 0x007FFFFF) | 0x3F800000
    m = plsc.bitcast(mbits, jnp.float32)
    big = m > 1.41421356
    m = jnp.where(big, m * 0.5, m)
    ef = e.astype(jnp.float32) + jnp.where(big, 1.0, 0.0)
    s = (m - 1.0) / (m + 1.0)
    z = s * s
    p = jnp.float32(1.0 / 9.0)
    p = p * z + jnp.float32(1.0 / 7.0)
    p = p * z + jnp.float32(0.2)
    p = p * z + jnp.float32(1.0 / 3.0)
    p = p * z + jnp.float32(1.0)
    return ef * jnp.float32(LN2) + 2.0 * s * p


def _make_glove(B, E):
    assert B % NW == 0 and E % L == 0
    b_per_w = B // NW
    CHUNK = 128
    n_chunks = b_per_w // CHUNK
    EJ = E // L

    mesh = plsc.VectorSubcoreMesh(
        core_axis_name="c", subcore_axis_name="s", num_cores=NC, num_subcores=NS
    )

    @functools.partial(
        pl.kernel,
        out_type=jax.ShapeDtypeStruct((NW, L), jnp.float32),
        mesh=mesh,
        compiler_params=pltpu.CompilerParams(needs_layout_passes=False),
        scratch_types=[
            pltpu.VMEM((b_per_w,), jnp.int32),
            pltpu.VMEM((b_per_w,), jnp.int32),
            pltpu.VMEM((b_per_w,), jnp.float32),
            pltpu.VMEM((2, CHUNK, E), jnp.float32),
            pltpu.VMEM((2, CHUNK, E), jnp.float32),
            pltpu.VMEM((2, CHUNK), jnp.float32),
            pltpu.VMEM((2, CHUNK), jnp.float32),
            pltpu.VMEM((L * L,), jnp.float32),
            pltpu.VMEM((L,), jnp.float32),
            pltpu.SemaphoreType.DMA((2,)),
        ],
    )
    def glove(fidx_hbm, cidx_hbm, cooc_hbm, ftab_hbm, ctab_hbm, fb_hbm, cb_hbm,
              out_hbm, fidx_v, cidx_v, cooc_v, frows_v, crows_v, fbv, cbv,
              pv, accv, sems):
        wid = lax.axis_index("s") * NC + lax.axis_index("c")
        base = wid * b_per_w

        pltpu.sync_copy(fidx_hbm.at[pl.ds(base, b_per_w)], fidx_v)
        pltpu.sync_copy(cidx_hbm.at[pl.ds(base, b_per_w)], cidx_v)
        pltpu.sync_copy(cooc_hbm.at[pl.ds(base, b_per_w)], cooc_v)

        def copies(k, buf):
            sem = sems.at[buf]
            fi = fidx_v.at[pl.ds(k * CHUNK, CHUNK)]
            ci = cidx_v.at[pl.ds(k * CHUNK, CHUNK)]
            return [
                pltpu.make_async_copy(ftab_hbm.at[fi], frows_v.at[buf], sem),
                pltpu.make_async_copy(ctab_hbm.at[ci], crows_v.at[buf], sem),
                pltpu.make_async_copy(fb_hbm.at[fi], fbv.at[buf], sem),
                pltpu.make_async_copy(cb_hbm.at[ci], cbv.at[buf], sem),
            ]

        def fire(k, buf):
            for cp in copies(k, buf):
                cp.start()

        def drain(k, buf):
            for cp in copies(k, buf):
                cp.wait()

        rid16 = lax.iota(jnp.int32, 16) * L
        fire(0, 0)
        acc0 = jnp.zeros((L,), jnp.float32)

        def chunk_body(k, acc):
            buf = k % 2
            drain(k, buf)

            @pl.when(k + 1 < n_chunks)
            def _():
                fire(k + 1, (k + 1) % 2)

            def group(g, acc):
                pbase = g * L
                for p in range(L):
                    row = pbase + p
                    prod0 = (frows_v[buf, row, pl.ds(0, L)]
                             * crows_v[buf, row, pl.ds(0, L)])
                    prod1 = (frows_v[buf, row, pl.ds(L, L)]
                             * crows_v[buf, row, pl.ds(L, L)])
                    for j in range(2, EJ, 2):
                        prod0 += (frows_v[buf, row, pl.ds(j * L, L)]
                                  * crows_v[buf, row, pl.ds(j * L, L)])
                        prod1 += (frows_v[buf, row, pl.ds((j + 1) * L, L)]
                                  * crows_v[buf, row, pl.ds((j + 1) * L, L)])
                    pv[pl.ds(p * L, L)] = prod0 + prod1
                dots0 = plsc.load_gather(pv, [rid16])
                dots1 = plsc.load_gather(pv, [rid16 + 1])
                for j in range(2, L, 2):
                    dots0 += plsc.load_gather(pv, [rid16 + j])
                    dots1 += plsc.load_gather(pv, [rid16 + j + 1])
                dots = dots0 + dots1
                f_b = fbv[buf, pl.ds(pbase, L)]
                c_b = cbv[buf, pl.ds(pbase, L)]
                x = cooc_v[pl.ds(k * CHUNK + pbase, L)]
                lnx = _ln(x)
                d = dots + f_b + c_b - lnx
                w = jnp.minimum(jnp.exp(ALPHA * lnx - W_BIAS), 1.0)
                return acc + w * d * d

            return lax.fori_loop(0, CHUNK // L, group, acc)

        acc = lax.fori_loop(0, n_chunks, chunk_body, acc0)

        accv[:] = acc
        pltpu.sync_copy(accv, out_hbm.at[wid])

    return glove


def kernel(focal_input, context_input, cooc_count, focal_table, context_table,
           focal_biases, context_biases):
    B = focal_input.shape[0]
    E = focal_table.shape[1]
    glove = _make_glove(B, E)
    partials = glove(
        focal_input.astype(jnp.int32), context_input.astype(jnp.int32),
        cooc_count, focal_table, context_table, focal_biases, context_biases)
    return jnp.sum(partials) / B
